# Initial kernel scaffold; baseline (speedup 1.0000x reference)
import jax
import jax.numpy as jnp
from jax.experimental import pallas as pl


def kernel(h, e_embed, edge_index, n_edges, W_msg, b_msg, W_upd, b_upd):
    raise NotImplementedError("write your pallas kernel here")


# speedup vs baseline: 1149.9423x; 1149.9423x over previous
import functools

import jax
import jax.numpy as jnp
from jax.experimental import pallas as pl
from jax.experimental.pallas import tpu as pltpu


def _mm_bias_body(x_ref, w_ref, b_ref, o_ref):
    o_ref[...] = (
        jnp.dot(x_ref[...], w_ref[...], preferred_element_type=jnp.float32)
        + b_ref[...]
    )


def _matmul_bias(x, w, b, bm):
    M, K = x.shape
    Hout = w.shape[1]
    return pl.pallas_call(
        _mm_bias_body,
        grid=(M // bm,),
        in_specs=[
            pl.BlockSpec((bm, K), lambda i: (i, 0)),
            pl.BlockSpec((K, Hout), lambda i: (0, 0)),
            pl.BlockSpec((1, Hout), lambda i: (0, 0)),
        ],
        out_specs=pl.BlockSpec((bm, Hout), lambda i: (i, 0)),
        out_shape=jax.ShapeDtypeStruct((M, Hout), jnp.float32),
    )(x, w, b)


def _upd_body(h_ref, a_ref, c_ref, w1_ref, w2_ref, b_ref, o_ref):
    cnt = jnp.maximum(c_ref[...], 1.0)
    agg = a_ref[...] / cnt
    acc = jnp.dot(h_ref[...], w1_ref[...], preferred_element_type=jnp.float32)
    acc = acc + jnp.dot(agg, w2_ref[...], preferred_element_type=jnp.float32)
    o_ref[...] = jnp.maximum(acc + b_ref[...], 0.0)


def _update_mlp(h2, aggsum, counts, w1, w2, b, bm):
    N, Hh = h2.shape
    return pl.pallas_call(
        _upd_body,
        grid=(N // bm,),
        in_specs=[
            pl.BlockSpec((bm, Hh), lambda i: (i, 0)),
            pl.BlockSpec((bm, Hh), lambda i: (i, 0)),
            pl.BlockSpec((bm, 1), lambda i: (i, 0)),
            pl.BlockSpec((Hh, Hh), lambda i: (0, 0)),
            pl.BlockSpec((Hh, Hh), lambda i: (0, 0)),
            pl.BlockSpec((1, Hh), lambda i: (0, 0)),
        ],
        out_specs=pl.BlockSpec((bm, Hh), lambda i: (i, 0)),
        out_shape=jax.ShapeDtypeStruct((N, Hh), jnp.float32),
    )(h2, aggsum, counts, w1, w2, b)


def kernel(h, e_embed, edge_index, n_edges, W_msg, b_msg, W_upd, b_upd):
    B, N, Hh = h.shape
    E = edge_index.shape[2]
    h2 = h[0]
    e2 = e_embed[0]
    src = edge_index[0, 0]
    dst = edge_index[0, 1]
    ne = n_edges[0, 0]

    W1 = W_msg[:, :Hh].T
    W2 = W_msg[:, Hh:].T
    Wu1 = W_upd[:, :Hh].T
    Wu2 = W_upd[:, Hh:].T

    zb = jnp.zeros((1, Hh), jnp.float32)
    P = _matmul_bias(h2, W1, zb, 2000)
    M2 = _matmul_bias(e2, W2, b_msg.reshape(1, Hh), 1280)

    G = jnp.take(P, src, axis=0)
    msg = jnp.maximum(G + M2, 0.0)
    valid = jnp.arange(E, dtype=jnp.int32) < ne
    dst2 = jnp.where(valid, dst, N)
    aggsum = jnp.zeros((N + 1, Hh), jnp.float32).at[dst2].add(msg)[:N]
    counts = jnp.zeros((N + 1,), jnp.float32).at[dst2].add(1.0)[:N]

    h_new = _update_mlp(h2, aggsum, counts[:, None], Wu1, Wu2,
                        b_upd.reshape(1, Hh), 2000)
    return h_new[None]


# speedup vs baseline: 1707.9150x; 1.4852x over previous
import functools

import jax
import jax.numpy as jnp
from jax import lax
from jax.experimental import pallas as pl
from jax.experimental.pallas import tpu as pltpu
from jax.experimental.pallas import tpu_sc as plsc

H = 256
HC = 128
NSC = 2
NTILE = 16
LANES = 16
CHUNK = 80
NACC = 10112


def _split_mm_body(x_ref, w_ref, b_ref, o_ref):
    o_ref[...] = (
        jnp.dot(x_ref[...], w_ref[...], preferred_element_type=jnp.float32)
        + b_ref[...]
    )


def _split_matmul(x, w, b, bm):
    M, K = x.shape
    nb = M // bm
    return pl.pallas_call(
        _split_mm_body,
        grid=(nb, NSC),
        in_specs=[
            pl.BlockSpec((bm, K), lambda i, j: (i, 0)),
            pl.BlockSpec((K, HC), lambda i, j: (0, j)),
            pl.BlockSpec((1, HC), lambda i, j: (0, j)),
        ],
        out_specs=pl.BlockSpec((bm, HC), lambda i, j: (j * nb + i, 0)),
        out_shape=jax.ShapeDtypeStruct((NSC * M, HC), jnp.float32),
    )(x, w, b)


def _cnt_body(d_ref, o_ref):
    i = pl.program_id(0)

    @pl.when(i == 0)
    def _():
        o_ref[...] = jnp.zeros_like(o_ref)

    d = d_ref[...]
    hi = d >> 7
    lo = d & 127
    cols = lax.broadcasted_iota(jnp.int32, (1, 128), 1)
    u = (hi == cols).astype(jnp.float32)
    v = (lo == cols).astype(jnp.float32)
    o_ref[...] += lax.dot_general(
        u, v, (((0,), (0,)), ((), ())), preferred_element_type=jnp.float32)


def _degree_counts(dst2, bk):
    E = dst2.shape[0]
    return pl.pallas_call(
        _cnt_body,
        grid=(E // bk,),
        in_specs=[pl.BlockSpec((bk, 1), lambda i: (i, 0))],
        out_specs=pl.BlockSpec((128, 128), lambda i: (0, 0)),
        out_shape=jax.ShapeDtypeStruct((128, 128), jnp.float32),
    )(dst2[:, None])


def _upd_body(h_ref, a0_ref, a1_ref, c_ref, w1_ref, w2_ref, b_ref, o_ref):
    cnt = jnp.maximum(c_ref[...], 1.0)
    agg = jnp.concatenate([a0_ref[...], a1_ref[...]], axis=1) / cnt
    acc = jnp.dot(h_ref[...], w1_ref[...], preferred_element_type=jnp.float32)
    acc = acc + jnp.dot(agg, w2_ref[...], preferred_element_type=jnp.float32)
    o_ref[...] = jnp.maximum(acc + b_ref[...], 0.0)


def _update_mlp(h2, omsg, counts, w1, w2, b, bm):
    N, Hh = h2.shape
    nb = N // bm
    return pl.pallas_call(
        _upd_body,
        grid=(nb,),
        in_specs=[
            pl.BlockSpec((bm, Hh), lambda i: (i, 0)),
            pl.BlockSpec((bm, HC), lambda i: (i, 0)),
            pl.BlockSpec((bm, HC), lambda i: (nb + i, 0)),
            pl.BlockSpec((bm, 1), lambda i: (i, 0)),
            pl.BlockSpec((Hh, Hh), lambda i: (0, 0)),
            pl.BlockSpec((Hh, Hh), lambda i: (0, 0)),
            pl.BlockSpec((1, Hh), lambda i: (0, 0)),
        ],
        out_specs=pl.BlockSpec((bm, Hh), lambda i: (i, 0)),
        out_shape=jax.ShapeDtypeStruct((N, Hh), jnp.float32),
    )(h2, omsg, omsg, counts, w1, w2, b)


def _make_sc_agg(N, E):
    Et = E // NTILE
    nch = Et // CHUNK
    rpt_acc = NACC // NTILE
    rpt_out = 624
    tail = N - NTILE * rpt_out
    mesh = plsc.VectorSubcoreMesh(core_axis_name="c", subcore_axis_name="s",
                                  num_cores=NSC, num_subcores=NTILE)

    @functools.partial(
        pl.kernel,
        out_type=jax.ShapeDtypeStruct((NSC * N, HC), jnp.float32),
        mesh=mesh,
        scratch_types=[
            pltpu.VMEM_SHARED((NACC, HC), jnp.float32),
            pltpu.VMEM((CHUNK,), jnp.int32),
            pltpu.VMEM((CHUNK,), jnp.int32),
            pltpu.VMEM((CHUNK, HC), jnp.float32),
            pltpu.VMEM((CHUNK, HC), jnp.float32),
            pltpu.SemaphoreType.DMA,
        ],
    )
    def sc_agg(p2, srcb, dst2, m2f, zacc, omsg,
               acc_sh, src_v, dst_v, m_v, g_v, sem):
        c = lax.axis_index("c")
        s = lax.axis_index("s")

        pltpu.sync_copy(zacc.at[pl.ds(s * rpt_acc, rpt_acc)],
                        acc_sh.at[pl.ds(s * rpt_acc, rpt_acc)])
        plsc.subcore_barrier()

        e0 = s * Et
        cE = c * E

        def chunk_body(q, carry):
            base = e0 + q * CHUNK
            pltpu.sync_copy(srcb.at[pl.ds(cE + base, CHUNK)], src_v)
            pltpu.sync_copy(dst2.at[pl.ds(base, CHUNK)], dst_v)
            pltpu.sync_copy(m2f.at[pl.ds(cE + base, CHUNK)], m_v)
            pltpu.async_copy(p2.at[src_v], g_v, sem).wait()

            def relu_row(r, carry2):
                for k in range(HC // LANES):
                    v = m_v[r, pl.ds(k * LANES, LANES)]
                    g = g_v[r, pl.ds(k * LANES, LANES)]
                    m_v[r, pl.ds(k * LANES, LANES)] = jnp.maximum(v + g, 0.0)
                return carry2

            lax.fori_loop(0, CHUNK, relu_row, 0, unroll=2)

            pltpu.sync_copy(m_v, acc_sh.at[dst_v], add=True)
            return carry

        lax.fori_loop(0, nch, chunk_body, 0)
        plsc.subcore_barrier()

        pltpu.sync_copy(acc_sh.at[pl.ds(s * rpt_out, rpt_out)],
                        omsg.at[pl.ds(c * N + s * rpt_out, rpt_out)])

        @pl.when(s == NTILE - 1)
        def _():
            t0 = NTILE * rpt_out
            pltpu.sync_copy(acc_sh.at[pl.ds(t0, tail)],
                            omsg.at[pl.ds(c * N + t0, tail)])

    return sc_agg


def kernel(h, e_embed, edge_index, n_edges, W_msg, b_msg, W_upd, b_upd):
    B, N, Hh = h.shape
    E = edge_index.shape[2]
    h2 = h[0]
    e2 = e_embed[0]
    src = edge_index[0, 0]
    dst = edge_index[0, 1]
    ne = n_edges[0, 0]

    W1 = W_msg[:, :Hh].T
    W2 = W_msg[:, Hh:].T
    Wu1 = W_upd[:, :Hh].T
    Wu2 = W_upd[:, Hh:].T

    zb = jnp.zeros((1, Hh), jnp.float32)
    p2 = _split_matmul(h2, W1, zb, 2000)
    m2f = _split_matmul(e2, W2, b_msg.reshape(1, Hh), 1600)

    srcb = jnp.concatenate([src, src + N])
    valid = jnp.arange(E, dtype=jnp.int32) < ne
    dst2 = jnp.where(valid, dst, N)

    zacc = jnp.zeros((NACC, HC), jnp.float32)
    omsg = _make_sc_agg(N, E)(p2, srcb, dst2, m2f, zacc)

    cnt_grid = _degree_counts(dst2, 1000)
    counts = cnt_grid.reshape(128 * 128)[:N, None]

    h_new = _update_mlp(h2, omsg, counts, Wu1, Wu2,
                        b_upd.reshape(1, Hh), 2000)
    return h_new[None]


# speedup vs baseline: 2473.9845x; 1.4485x over previous
import functools

import jax
import jax.numpy as jnp
from jax import lax
from jax.experimental import pallas as pl
from jax.experimental.pallas import tpu as pltpu
from jax.experimental.pallas import tpu_sc as plsc

H = 256
HC = 128
NSC = 2
NTILE = 16
LANES = 16
CHUNK = 40
NACC = 10112


def _split_mm_body(x_ref, w_ref, b_ref, o_ref):
    o_ref[...] = (
        jnp.dot(x_ref[...], w_ref[...], preferred_element_type=jnp.float32)
        + b_ref[...]
    )


def _split_matmul(x, w, b, bm):
    M, K = x.shape
    nb = M // bm
    return pl.pallas_call(
        _split_mm_body,
        grid=(nb, NSC),
        in_specs=[
            pl.BlockSpec((bm, K), lambda i, j: (i, 0)),
            pl.BlockSpec((K, HC), lambda i, j: (0, j)),
            pl.BlockSpec((1, HC), lambda i, j: (0, j)),
        ],
        out_specs=pl.BlockSpec((bm, HC), lambda i, j: (j * nb + i, 0)),
        out_shape=jax.ShapeDtypeStruct((NSC * M, HC), jnp.float32),
    )(x, w, b)


def _cnt_body(d_ref, o_ref):
    i = pl.program_id(0)

    @pl.when(i == 0)
    def _():
        o_ref[...] = jnp.zeros_like(o_ref)

    d = d_ref[...]
    hi = d >> 7
    lo = d & 127
    cols = lax.broadcasted_iota(jnp.int32, (1, 128), 1)
    u = (hi == cols).astype(jnp.float32)
    v = (lo == cols).astype(jnp.float32)
    o_ref[...] += lax.dot_general(
        u, v, (((0,), (0,)), ((), ())), preferred_element_type=jnp.float32)


def _degree_counts(dst2, bk):
    E = dst2.shape[0]
    return pl.pallas_call(
        _cnt_body,
        grid=(E // bk,),
        in_specs=[pl.BlockSpec((bk, 1), lambda i: (i, 0))],
        out_specs=pl.BlockSpec((128, 128), lambda i: (0, 0)),
        out_shape=jax.ShapeDtypeStruct((128, 128), jnp.float32),
    )(dst2[:, None])


def _upd_body(h_ref, a0_ref, a1_ref, c_ref, w1_ref, w2_ref, b_ref, o_ref):
    cnt = jnp.maximum(c_ref[...], 1.0)
    agg = jnp.concatenate([a0_ref[...], a1_ref[...]], axis=1) / cnt
    acc = jnp.dot(h_ref[...], w1_ref[...], preferred_element_type=jnp.float32)
    acc = acc + jnp.dot(agg, w2_ref[...], preferred_element_type=jnp.float32)
    o_ref[...] = jnp.maximum(acc + b_ref[...], 0.0)


def _update_mlp(h2, omsg, counts, w1, w2, b, bm):
    N, Hh = h2.shape
    nb = N // bm
    return pl.pallas_call(
        _upd_body,
        grid=(nb,),
        in_specs=[
            pl.BlockSpec((bm, Hh), lambda i: (i, 0)),
            pl.BlockSpec((bm, HC), lambda i: (i, 0)),
            pl.BlockSpec((bm, HC), lambda i: (nb + i, 0)),
            pl.BlockSpec((bm, 1), lambda i: (i, 0)),
            pl.BlockSpec((Hh, Hh), lambda i: (0, 0)),
            pl.BlockSpec((Hh, Hh), lambda i: (0, 0)),
            pl.BlockSpec((1, Hh), lambda i: (0, 0)),
        ],
        out_specs=pl.BlockSpec((bm, Hh), lambda i: (i, 0)),
        out_shape=jax.ShapeDtypeStruct((N, Hh), jnp.float32),
    )(h2, omsg, omsg, counts, w1, w2, b)


def _make_sc_agg(N, E):
    Et = E // NTILE
    nch = Et // CHUNK
    rpt_acc = NACC // NTILE
    rpt_out = 624
    tail = N - NTILE * rpt_out
    mesh = plsc.VectorSubcoreMesh(core_axis_name="c", subcore_axis_name="s",
                                  num_cores=NSC, num_subcores=NTILE)

    @functools.partial(
        pl.kernel,
        out_type=jax.ShapeDtypeStruct((NSC * N, HC), jnp.float32),
        mesh=mesh,
        scratch_types=[
            pltpu.VMEM_SHARED((NACC, HC), jnp.float32),
            pltpu.VMEM((Et,), jnp.int32),
            [pltpu.VMEM((CHUNK,), jnp.int32)] * 2,
            [pltpu.VMEM((CHUNK, HC), jnp.float32)] * 2,
            [pltpu.VMEM((CHUNK, HC), jnp.float32)] * 2,
            [pltpu.SemaphoreType.DMA] * 6,
        ],
    )
    def sc_agg(p2, srcb, dst2, m2f, zacc, omsg,
               acc_sh, src_all, dst_bufs, m_bufs, g_bufs, sems):
        c = lax.axis_index("c")
        s = lax.axis_index("s")

        pltpu.sync_copy(zacc.at[pl.ds(s * rpt_acc, rpt_acc)],
                        acc_sh.at[pl.ds(s * rpt_acc, rpt_acc)])

        e0 = s * Et
        cE = c * E

        pltpu.sync_copy(srcb.at[pl.ds(cE + e0, Et)], src_all)
        plsc.subcore_barrier()

        def start(q, b):
            pltpu.async_copy(m2f.at[pl.ds(cE + e0 + q * CHUNK, CHUNK)],
                             m_bufs[b], sems[b])
            pltpu.async_copy(p2.at[src_all.at[pl.ds(q * CHUNK, CHUNK)]],
                             g_bufs[b], sems[2 + b])
            pltpu.async_copy(dst2.at[pl.ds(e0 + q * CHUNK, CHUNK)],
                             dst_bufs[b], sems[4 + b])

        def finish(q, b):
            pltpu.make_async_copy(m2f.at[pl.ds(cE + e0 + q * CHUNK, CHUNK)],
                                  m_bufs[b], sems[b]).wait()
            pltpu.make_async_copy(p2.at[src_all.at[pl.ds(q * CHUNK, CHUNK)]],
                                  g_bufs[b], sems[2 + b]).wait()
            pltpu.make_async_copy(dst2.at[pl.ds(e0 + q * CHUNK, CHUNK)],
                                  dst_bufs[b], sems[4 + b]).wait()
            m_v, g_v = m_bufs[b], g_bufs[b]

            def relu_row(r, carry2):
                for k in range(HC // LANES):
                    v = m_v[r, pl.ds(k * LANES, LANES)]
                    g = g_v[r, pl.ds(k * LANES, LANES)]
                    m_v[r, pl.ds(k * LANES, LANES)] = jnp.maximum(v + g, 0.0)
                return carry2

            lax.fori_loop(0, CHUNK, relu_row, 0, unroll=2)
            pltpu.sync_copy(m_v, acc_sh.at[dst_bufs[b]], add=True)

        start(0, 0)

        def outer(q2, carry):
            q = q2 * 2
            start(q + 1, 1)
            finish(q, 0)
            start(q + 2, 0)
            finish(q + 1, 1)
            return carry

        lax.fori_loop(0, (nch - 1) // 2, outer, 0)
        if nch % 2 == 1:
            finish(nch - 1, 0)
        else:
            start(nch - 1, 1)
            finish(nch - 2, 0)
            finish(nch - 1, 1)
        plsc.subcore_barrier()

        pltpu.sync_copy(acc_sh.at[pl.ds(s * rpt_out, rpt_out)],
                        omsg.at[pl.ds(c * N + s * rpt_out, rpt_out)])

        @pl.when(s == NTILE - 1)
        def _():
            t0 = NTILE * rpt_out
            pltpu.sync_copy(acc_sh.at[pl.ds(t0, tail)],
                            omsg.at[pl.ds(c * N + t0, tail)])

    return sc_agg


def kernel(h, e_embed, edge_index, n_edges, W_msg, b_msg, W_upd, b_upd):
    B, N, Hh = h.shape
    E = edge_index.shape[2]
    h2 = h[0]
    e2 = e_embed[0]
    src = edge_index[0, 0]
    dst = edge_index[0, 1]
    ne = n_edges[0, 0]

    W1 = W_msg[:, :Hh].T
    W2 = W_msg[:, Hh:].T
    Wu1 = W_upd[:, :Hh].T
    Wu2 = W_upd[:, Hh:].T

    zb = jnp.zeros((1, Hh), jnp.float32)
    p2 = _split_matmul(h2, W1, zb, 2000)
    m2f = _split_matmul(e2, W2, b_msg.reshape(1, Hh), 1600)

    srcb = jnp.concatenate([src, src + N])
    valid = jnp.arange(E, dtype=jnp.int32) < ne
    dst2 = jnp.where(valid, dst, N)

    zacc = jnp.zeros((NACC, HC), jnp.float32)
    omsg = _make_sc_agg(N, E)(p2, srcb, dst2, m2f, zacc)

    cnt_grid = _degree_counts(dst2, 1000)
    counts = cnt_grid.reshape(128 * 128)[:N, None]

    h_new = _update_mlp(h2, omsg, counts, Wu1, Wu2,
                        b_upd.reshape(1, Hh), 2000)
    return h_new[None]


# speedup vs baseline: 2513.8889x; 1.0161x over previous
import functools

import jax
import jax.numpy as jnp
from jax import lax
from jax.experimental import pallas as pl
from jax.experimental.pallas import tpu as pltpu
from jax.experimental.pallas import tpu_sc as plsc

H = 256
HC = 128
NSC = 2
NTILE = 16
LANES = 16
CHUNK = 40
NACC = 10112


def _split_mm_body(x_ref, w_ref, b_ref, o_ref):
    o_ref[...] = (
        jnp.dot(x_ref[...], w_ref[...], preferred_element_type=jnp.float32)
        + b_ref[...]
    )


def _split_matmul(x, w, b, bm):
    M, K = x.shape
    nb = M // bm
    return pl.pallas_call(
        _split_mm_body,
        grid=(nb, NSC),
        in_specs=[
            pl.BlockSpec((bm, K), lambda i, j: (i, 0)),
            pl.BlockSpec((K, HC), lambda i, j: (0, j)),
            pl.BlockSpec((1, HC), lambda i, j: (0, j)),
        ],
        out_specs=pl.BlockSpec((bm, HC), lambda i, j: (j * nb + i, 0)),
        out_shape=jax.ShapeDtypeStruct((NSC * M, HC), jnp.float32),
    )(x, w, b)


def _cnt_body(d_ref, o_ref):
    i = pl.program_id(0)

    @pl.when(i == 0)
    def _():
        o_ref[...] = jnp.zeros_like(o_ref)

    d = d_ref[...]
    hi = d >> 7
    lo = d & 127
    cols = lax.broadcasted_iota(jnp.int32, (1, 128), 1)
    u = (hi == cols).astype(jnp.float32)
    v = (lo == cols).astype(jnp.float32)
    o_ref[...] += lax.dot_general(
        u, v, (((0,), (0,)), ((), ())), preferred_element_type=jnp.float32)


def _degree_counts(dst2, bk):
    E = dst2.shape[0]
    return pl.pallas_call(
        _cnt_body,
        grid=(E // bk,),
        in_specs=[pl.BlockSpec((bk, 1), lambda i: (i, 0))],
        out_specs=pl.BlockSpec((128, 128), lambda i: (0, 0)),
        out_shape=jax.ShapeDtypeStruct((128, 128), jnp.float32),
    )(dst2[:, None])


def _upd_body(h_ref, a0_ref, a1_ref, c_ref, w1_ref, w2_ref, b_ref, o_ref):
    cnt = jnp.maximum(c_ref[...], 1.0)
    agg = jnp.concatenate([a0_ref[...], a1_ref[...]], axis=1) / cnt
    acc = jnp.dot(h_ref[...], w1_ref[...], preferred_element_type=jnp.float32)
    acc = acc + jnp.dot(agg, w2_ref[...], preferred_element_type=jnp.float32)
    o_ref[...] = jnp.maximum(acc + b_ref[...], 0.0)


def _update_mlp(h2, omsg, counts, w1, w2, b, bm):
    N, Hh = h2.shape
    nb = N // bm
    return pl.pallas_call(
        _upd_body,
        grid=(nb,),
        in_specs=[
            pl.BlockSpec((bm, Hh), lambda i: (i, 0)),
            pl.BlockSpec((bm, HC), lambda i: (i, 0)),
            pl.BlockSpec((bm, HC), lambda i: (nb + i, 0)),
            pl.BlockSpec((bm, 1), lambda i: (i, 0)),
            pl.BlockSpec((Hh, Hh), lambda i: (0, 0)),
            pl.BlockSpec((Hh, Hh), lambda i: (0, 0)),
            pl.BlockSpec((1, Hh), lambda i: (0, 0)),
        ],
        out_specs=pl.BlockSpec((bm, Hh), lambda i: (i, 0)),
        out_shape=jax.ShapeDtypeStruct((N, Hh), jnp.float32),
    )(h2, omsg, omsg, counts, w1, w2, b)


def _make_sc_agg(N, E):
    Et = E // NTILE
    nch = Et // CHUNK
    rpt_acc = NACC // NTILE
    rpt_out = 624
    tail = N - NTILE * rpt_out
    mesh = plsc.VectorSubcoreMesh(core_axis_name="c", subcore_axis_name="s",
                                  num_cores=NSC, num_subcores=NTILE)

    @functools.partial(
        pl.kernel,
        out_type=jax.ShapeDtypeStruct((NSC * N, HC), jnp.float32),
        mesh=mesh,
        scratch_types=[
            pltpu.VMEM_SHARED((NACC, HC), jnp.float32),
            pltpu.VMEM((Et,), jnp.int32),
            [pltpu.VMEM((CHUNK,), jnp.int32)] * 2,
            [pltpu.VMEM((CHUNK,), jnp.int32)] * 2,
            [pltpu.VMEM((CHUNK, HC), jnp.float32)] * 2,
            [pltpu.VMEM((CHUNK, HC), jnp.float32)] * 2,
            [pltpu.VMEM((CHUNK, HC), jnp.float32)] * 2,
            [pltpu.SemaphoreType.DMA] * 8,
        ],
    )
    def sc_agg(p2, srcb, dst2, m2f, zacc, omsg,
               acc_sh, src_all, dst_bufs, sdst_bufs, m_bufs, g_bufs, o_bufs,
               sems):
        c = lax.axis_index("c")
        s = lax.axis_index("s")

        pltpu.sync_copy(zacc.at[pl.ds(s * rpt_acc, rpt_acc)],
                        acc_sh.at[pl.ds(s * rpt_acc, rpt_acc)])

        e0 = s * Et
        cE = c * E

        pltpu.sync_copy(srcb.at[pl.ds(cE + e0, Et)], src_all)
        plsc.subcore_barrier()

        def start(q, b):
            pltpu.async_copy(m2f.at[pl.ds(cE + e0 + q * CHUNK, CHUNK)],
                             m_bufs[b], sems[b])
            pltpu.async_copy(p2.at[src_all.at[pl.ds(q * CHUNK, CHUNK)]],
                             g_bufs[b], sems[2 + b])
            pltpu.async_copy(dst2.at[pl.ds(e0 + q * CHUNK, CHUNK)],
                             dst_bufs[b], sems[4 + b])

        def finish(q, b, first=False):
            pltpu.make_async_copy(m2f.at[pl.ds(cE + e0 + q * CHUNK, CHUNK)],
                                  m_bufs[b], sems[b]).wait()
            pltpu.make_async_copy(p2.at[src_all.at[pl.ds(q * CHUNK, CHUNK)]],
                                  g_bufs[b], sems[2 + b]).wait()
            pltpu.make_async_copy(dst2.at[pl.ds(e0 + q * CHUNK, CHUNK)],
                                  dst_bufs[b], sems[4 + b]).wait()
            if not first:
                pltpu.make_async_copy(o_bufs[b], acc_sh.at[sdst_bufs[b]],
                                      sems[6 + b]).wait()
            m_v, g_v, o_v = m_bufs[b], g_bufs[b], o_bufs[b]
            for k in (0, 16, CHUNK - LANES):
                sdst_bufs[b][pl.ds(k, LANES)] = dst_bufs[b][pl.ds(k, LANES)]

            def relu_row(r, carry2):
                for k in range(HC // LANES):
                    v = m_v[r, pl.ds(k * LANES, LANES)]
                    g = g_v[r, pl.ds(k * LANES, LANES)]
                    o_v[r, pl.ds(k * LANES, LANES)] = jnp.maximum(v + g, 0.0)
                return carry2

            lax.fori_loop(0, CHUNK, relu_row, 0, unroll=4)
            pltpu.async_copy(o_v, acc_sh.at[sdst_bufs[b]], sems[6 + b],
                             add=True)

        start(0, 0)
        start(1, 1)
        finish(0, 0, first=True)
        start(2, 0)
        finish(1, 1, first=True)
        start(3, 1)

        def outer(q2, carry):
            q = q2 * 2
            finish(q, 0)
            start(q + 2, 0)
            finish(q + 1, 1)
            start(q + 3, 1)
            return carry

        lax.fori_loop(1, (nch - 2) // 2, outer, 0)
        finish(nch - 2, 0)
        finish(nch - 1, 1)
        pltpu.make_async_copy(o_bufs[0], acc_sh.at[sdst_bufs[0]],
                              sems[6]).wait()
        pltpu.make_async_copy(o_bufs[1], acc_sh.at[sdst_bufs[1]],
                              sems[7]).wait()
        plsc.subcore_barrier()

        pltpu.sync_copy(acc_sh.at[pl.ds(s * rpt_out, rpt_out)],
                        omsg.at[pl.ds(c * N + s * rpt_out, rpt_out)])

        @pl.when(s == NTILE - 1)
        def _():
            t0 = NTILE * rpt_out
            pltpu.sync_copy(acc_sh.at[pl.ds(t0, tail)],
                            omsg.at[pl.ds(c * N + t0, tail)])

    return sc_agg


def kernel(h, e_embed, edge_index, n_edges, W_msg, b_msg, W_upd, b_upd):
    B, N, Hh = h.shape
    E = edge_index.shape[2]
    h2 = h[0]
    e2 = e_embed[0]
    src = edge_index[0, 0]
    dst = edge_index[0, 1]
    ne = n_edges[0, 0]

    W1 = W_msg[:, :Hh].T
    W2 = W_msg[:, Hh:].T
    Wu1 = W_upd[:, :Hh].T
    Wu2 = W_upd[:, Hh:].T

    zb = jnp.zeros((1, Hh), jnp.float32)
    p2 = _split_matmul(h2, W1, zb, 2000)
    m2f = _split_matmul(e2, W2, b_msg.reshape(1, Hh), 1600)

    srcb = jnp.concatenate([src, src + N])
    valid = jnp.arange(E, dtype=jnp.int32) < ne
    dst2 = jnp.where(valid, dst, N)

    zacc = jnp.zeros((NACC, HC), jnp.float32)
    omsg = _make_sc_agg(N, E)(p2, srcb, dst2, m2f, zacc)

    cnt_grid = _degree_counts(dst2, 1000)
    counts = cnt_grid.reshape(128 * 128)[:N, None]

    h_new = _update_mlp(h2, omsg, counts, Wu1, Wu2,
                        b_upd.reshape(1, Hh), 2000)
    return h_new[None]
